# baseline (speedup 1.0000x reference)
import functools

import jax
import jax.numpy as jnp
from jax import lax
from jax.experimental import pallas as pl
from jax.experimental.pallas import tpu as pltpu
from jax.experimental.pallas import tpu_sc as plsc


_NC = 2
_NS = 16
_NW = _NC * _NS
_LANES = 16
_LS = 2048
_CH = 32
_BL = 1024


def _make_sc_add(b, lfull, ls, d):
    lpw = ls // _NW
    nch = lpw // _CH
    nsteps = nch * b
    nvec = d // _LANES
    mesh = plsc.VectorSubcoreMesh(core_axis_name="c", subcore_axis_name="s")

    @functools.partial(
        pl.kernel,
        out_type=jax.ShapeDtypeStruct((b * ls, d), jnp.float32),
        mesh=mesh,
        scratch_types=[
            pltpu.VMEM((_CH, d), jnp.float32),
            pltpu.VMEM((_CH, d), jnp.float32),
            pltpu.VMEM((_CH, d), jnp.float32),
            pltpu.VMEM((_CH, d), jnp.float32),
            pltpu.SemaphoreType.DMA,
            pltpu.SemaphoreType.DMA,
            pltpu.SemaphoreType.DMA,
            pltpu.SemaphoreType.DMA,
            pltpu.SemaphoreType.DMA,
            pltpu.SemaphoreType.DMA,
        ],
    )
    def sc_add(x_hbm, table_hbm, out_hbm,
               xb0, xb1, tb0, tb1, sx0, sx1, st0, st1, so0, so1):
        bufs = (xb0, xb1)
        tbufs = (tb0, tb1)
        sxs = (sx0, sx1)
        sts = (st0, st1)
        sos = (so0, so1)
        cid = lax.axis_index("c")
        sid = lax.axis_index("s")
        wid = cid * _NS + sid
        tbase = wid * lpw

        def t_slice(i):
            return table_hbm.at[pl.ds(tbase + i * _CH, _CH)]

        def x_slice(i, bi):
            return x_hbm.at[pl.ds(bi * lfull + tbase + i * _CH, _CH)]

        def o_slice(i, bi):
            return out_hbm.at[pl.ds(bi * ls + tbase + i * _CH, _CH)]

        pltpu.async_copy(t_slice(0), tbufs[0], sts[0])
        pltpu.async_copy(t_slice(1), tbufs[1], sts[1])
        pltpu.async_copy(x_slice(0, 0), bufs[0], sxs[0])

        def iter_body(i2, _):
            for ip in range(2):
                i = 2 * i2 + ip
                pltpu.make_async_copy(t_slice(i), tbufs[ip], sts[ip]).wait()
                for bi in range(b):
                    p = bi % 2
                    s = i * b + bi
                    xb = bufs[p]
                    pltpu.make_async_copy(
                        x_slice(i, bi), xb, sxs[p]).wait()
                    nbi = (bi + 1) % b
                    ni = i + (1 if bi == b - 1 else 0)

                    @pl.when(s + 1 < nsteps)
                    def _():
                        @pl.when(s >= 1)
                        def _():
                            pltpu.make_async_copy(
                                bufs[1 - p], o_slice(ni, nbi),
                                sos[1 - p]).wait()
                        pltpu.async_copy(
                            x_slice(ni, nbi), bufs[1 - p], sxs[1 - p])

                    tb = tbufs[ip]

                    @plsc.parallel_loop(0, _CH, step=1)
                    def add_body(r, xb=xb, tb=tb):
                        for c in range(nvec):
                            plsc.addupdate(
                                xb.at[r].at[pl.ds(c * _LANES, _LANES)],
                                tb[r, pl.ds(c * _LANES, _LANES)])

                    pltpu.async_copy(xb, o_slice(i, bi), sos[p])

                @pl.when(i + 2 < nch)
                def _():
                    pltpu.async_copy(t_slice(i + 2), tbufs[ip], sts[ip])
            return 0

        lax.fori_loop(0, nch // 2, iter_body, 0)
        pltpu.make_async_copy(bufs[0], o_slice(nch - 1, b - 2), sos[0]).wait()
        pltpu.make_async_copy(bufs[1], o_slice(nch - 1, b - 1), sos[1]).wait()

    return sc_add


def _tc_add_kernel(x_ref, t_ref, o_ref):
    o_ref[...] = x_ref[...] + t_ref[...]


def _tc_add(x, table, ls):
    B, L, D = x.shape
    ltc = L - ls
    off = ls // _BL
    grid = (ltc // _BL,)
    return pl.pallas_call(
        _tc_add_kernel,
        grid=grid,
        in_specs=[
            pl.BlockSpec((B, _BL, D), lambda i: (0, off + i, 0)),
            pl.BlockSpec((_BL, D), lambda i: (off + i, 0)),
        ],
        out_specs=pl.BlockSpec((B, _BL, D), lambda i: (0, i, 0)),
        out_shape=jax.ShapeDtypeStruct((B, ltc, D), x.dtype),
    )(x, table)


def kernel(x, table):
    B, L, D = x.shape
    out_sc = _make_sc_add(B, L, _LS, D)(x.reshape(B * L, D), table)
    out_tc = _tc_add(x, table, _LS)
    return jnp.concatenate([out_sc.reshape(B, _LS, D), out_tc], axis=1)
